# baseline (speedup 1.0000x reference)
import functools

import jax
import jax.numpy as jnp
from jax import lax
from jax.experimental import pallas as pl
from jax.experimental.pallas import tpu as pltpu
from jax.experimental.pallas import tpu_sc as plsc

_RB = 512


def _prep_kernel(x_ref, w_ref, a1_ref, a2_ref, wh_ref, s1_ref, s2_ref):
    wh = jnp.dot(x_ref[...], w_ref[...], preferred_element_type=jnp.float32)
    wh_ref[...] = wh
    s1_ref[...] = jnp.dot(wh, a1_ref[...], preferred_element_type=jnp.float32)
    s2_ref[...] = lax.dot_general(
        a2_ref[...], wh, (((0,), (1,)), ((), ())),
        preferred_element_type=jnp.float32)


def _gat_kernel(adj_ref, s1_ref, s2_ref, wh_ref, out_ref, *, half):
    e = s1_ref[...] + s2_ref[...]
    e = jnp.maximum(e, 0.2 * e)
    c = jnp.exp(adj_ref[...] * e)
    z = jnp.sum(c, axis=1, keepdims=True)
    acc = jnp.dot(c, wh_ref[...], preferred_element_type=jnp.float32)
    h = acc / z
    out_ref[...] = 0.5 * (h[:, :half] + h[:, half:])


def _sc_stream_kernel(adj_hbm, out_hbm, buf0, buf1, sem0, sem1):
    wid = lax.axis_index("s") * 2 + lax.axis_index("c")
    rows_per_w = 128
    chunk = 8
    n_chunks = rows_per_w // chunk
    base = wid * rows_per_w
    cps = []
    for ch in range(n_chunks):
        buf = buf0 if ch % 2 == 0 else buf1
        sem = sem0 if ch % 2 == 0 else sem1
        if ch >= 2:
            cps[ch - 2].wait()
        cps.append(
            pltpu.async_copy(adj_hbm.at[pl.ds(base + ch * chunk, chunk)],
                             buf, sem))
    cps[-2].wait()
    cps[-1].wait()
    pltpu.sync_copy(buf1.at[pl.ds(0, 1)], out_hbm.at[pl.ds(wid, 1)])


def kernel(x, adj, W, a):
    n, _ = x.shape
    nc = adj.shape[1]
    out_f = W.shape[1]
    half = out_f // 2
    a1 = a[:out_f]
    a2 = a[out_f:]

    wh, s1, s2 = pl.pallas_call(
        _prep_kernel,
        out_shape=[
            jax.ShapeDtypeStruct((n, out_f), jnp.float32),
            jax.ShapeDtypeStruct((n, 1), jnp.float32),
            jax.ShapeDtypeStruct((1, n), jnp.float32),
        ],
    )(x, W, a1, a2)

    sc_out = pl.kernel(
        _sc_stream_kernel,
        out_type=jax.ShapeDtypeStruct((32, nc), jnp.float32),
        mesh=plsc.VectorSubcoreMesh(core_axis_name="c", subcore_axis_name="s"),
        scratch_types=[
            pltpu.VMEM((8, nc), jnp.float32),
            pltpu.VMEM((8, nc), jnp.float32),
            pltpu.SemaphoreType.DMA,
            pltpu.SemaphoreType.DMA,
        ],
    )(adj)

    out = pl.pallas_call(
        functools.partial(_gat_kernel, half=half),
        grid=(n // _RB,),
        in_specs=[
            pl.BlockSpec((_RB, nc), lambda i: (i, 0)),
            pl.BlockSpec((_RB, 1), lambda i: (i, 0)),
            pl.BlockSpec((1, nc), lambda i: (0, 0)),
            pl.BlockSpec((n, out_f), lambda i: (0, 0)),
        ],
        out_specs=pl.BlockSpec((_RB, half), lambda i: (i, 0)),
        out_shape=jax.ShapeDtypeStruct((n, half), jnp.float32),
    )(adj, s1, s2, wh)
    return out + 0.0 * sc_out[0, :half]


# speedup vs baseline: 1.9248x; 1.9248x over previous
import functools

import jax
import jax.numpy as jnp
from jax import lax
from jax.experimental import pallas as pl
from jax.experimental.pallas import tpu as pltpu

_RB = 512


def _gat_kernel(x_ref, w_ref, a1_ref, a2_ref, adj_ref, out_ref,
                wh_ref, s1_ref, s2_ref, *, half, rb):
    i = pl.program_id(0)

    @pl.when(i == 0)
    def _prep():
        wh = jnp.dot(x_ref[...], w_ref[...],
                     preferred_element_type=jnp.float32)
        wh_ref[...] = wh
        s1_ref[...] = jnp.dot(wh, a1_ref[...],
                              preferred_element_type=jnp.float32)
        s2_ref[...] = lax.dot_general(
            a2_ref[...], wh, (((0,), (1,)), ((), ())),
            preferred_element_type=jnp.float32)

    e = s1_ref[pl.ds(i * rb, rb), :] + s2_ref[...]
    e = jnp.maximum(e, 0.2 * e)
    c = jnp.exp(adj_ref[...] * e)
    z = jnp.sum(c, axis=1, keepdims=True)
    acc = jnp.dot(c, wh_ref[...], preferred_element_type=jnp.float32)
    h = acc / z
    out_ref[...] = 0.5 * (h[:, :half] + h[:, half:])


def kernel(x, adj, W, a):
    n, in_f = x.shape
    nc = adj.shape[1]
    out_f = W.shape[1]
    half = out_f // 2
    a1 = a[:out_f]
    a2 = a[out_f:]

    out = pl.pallas_call(
        functools.partial(_gat_kernel, half=half, rb=_RB),
        grid=(n // _RB,),
        in_specs=[
            pl.BlockSpec((n, in_f), lambda i: (0, 0)),
            pl.BlockSpec((in_f, out_f), lambda i: (0, 0)),
            pl.BlockSpec((out_f, 1), lambda i: (0, 0)),
            pl.BlockSpec((out_f, 1), lambda i: (0, 0)),
            pl.BlockSpec((_RB, nc), lambda i: (i, 0)),
        ],
        out_specs=pl.BlockSpec((_RB, half), lambda i: (i, 0)),
        out_shape=jax.ShapeDtypeStruct((n, half), jnp.float32),
        scratch_shapes=[
            pltpu.VMEM((n, out_f), jnp.float32),
            pltpu.VMEM((n, 1), jnp.float32),
            pltpu.VMEM((1, nc), jnp.float32),
        ],
    )(x, W, a1, a2, adj)
    return out
